# Initial kernel scaffold; baseline (speedup 1.0000x reference)
import jax
import jax.numpy as jnp
from jax.experimental import pallas as pl


def kernel(user, item, user_gender, user_age, user_occupation, user_neighbor, item_neighbor, W_user_gmf, W_item_gmf, W_user_sage, W_item_sage, W_gender, W_age, W_occ, W_all, b_all, W_cu, b_cu, W_ci, b_ci, W_p1, b_p1, W_p2, b_p2):
    raise NotImplementedError("write your pallas kernel here")


# speedup vs baseline: 3.0166x; 3.0166x over previous
import functools

import jax
import jax.numpy as jnp
from jax import lax
from jax.experimental import pallas as pl
from jax.experimental.pallas import tpu as pltpu
from jax.experimental.pallas import tpu_sc as plsc

B = 4096
F = 128
NB = 20
NC = 2
NS = 16
NW = NC * NS
BPW = B // NW
TB = 512
NT = B // TB

def _sc_gather_body(user, item, un_t, in_t, w_user_gmf, w_item_gmf, w_user_sage,
                    w_item_sage, uid_out, item_out, usage_out, isage_out,
                    idq, idxn, buf0, buf1, acc, sem0, sem1):
    wid = lax.axis_index("s") * NC + lax.axis_index("c")
    base = wid * BPW

    pltpu.sync_copy(user.at[pl.ds(base, BPW)], idq)
    pltpu.async_copy(w_user_gmf.at[idq], buf0, sem0).wait()
    pltpu.sync_copy(buf0, uid_out.at[pl.ds(base, BPW)])
    pltpu.sync_copy(item.at[pl.ds(base, BPW)], idq)
    pltpu.async_copy(w_item_gmf.at[idq], buf0, sem0).wait()
    pltpu.sync_copy(buf0, item_out.at[pl.ds(base, BPW)])

    def side(idx_hbm, table, out):
        pltpu.sync_copy(idx_hbm.at[wid], idxn)
        bufs = (buf0, buf1)
        sems = (sem0, sem1)
        cps = {0: pltpu.async_copy(table.at[idxn.at[0]], bufs[0], sems[0])}
        for k in range(NB):
            if k + 1 < NB:
                cps[k + 1] = pltpu.async_copy(
                    table.at[idxn.at[k + 1]], bufs[(k + 1) % 2], sems[(k + 1) % 2])
            cps[k].wait()
            b = bufs[k % 2]
            if k == 0:
                def body0(r, carry):
                    for v in range(F // 16):
                        acc[r, pl.ds(v * 16, 16)] = b[r, pl.ds(v * 16, 16)]
                    return carry
                lax.fori_loop(0, BPW, body0, 0)
            else:
                def bodyk(r, carry, b=b):
                    for v in range(F // 16):
                        plsc.addupdate(acc.at[r, pl.ds(v * 16, 16)],
                                       b[r, pl.ds(v * 16, 16)])
                    return carry
                lax.fori_loop(0, BPW, bodyk, 0)
        pltpu.sync_copy(acc, out.at[pl.ds(base, BPW)])

    side(un_t, w_item_sage, usage_out)
    side(in_t, w_user_sage, isage_out)


@functools.lru_cache(maxsize=1)
def _sc_gather():
    return pl.kernel(
        _sc_gather_body,
        out_type=(
            jax.ShapeDtypeStruct((B, F), jnp.float32),
            jax.ShapeDtypeStruct((B, F), jnp.float32),
            jax.ShapeDtypeStruct((B, F), jnp.float32),
            jax.ShapeDtypeStruct((B, F), jnp.float32),
        ),
        mesh=plsc.VectorSubcoreMesh(core_axis_name="c", subcore_axis_name="s"),
        scratch_types=[
            pltpu.VMEM((BPW,), jnp.int32),
            pltpu.VMEM((NB, BPW), jnp.int32),
            pltpu.VMEM((BPW, F), jnp.float32),
            pltpu.VMEM((BPW, F), jnp.float32),
            pltpu.VMEM((BPW, F), jnp.float32),
            pltpu.SemaphoreType.DMA,
            pltpu.SemaphoreType.DMA,
        ],
    )


def _tc_mlp_body(uid_ref, item_ref, us_ref, is_ref, g_ref, a_ref, o_ref,
                 wall_ref, wsm_ref, wcu_ref, wci_ref, ball_ref, bcu_ref,
                 bci_ref, wp1t_ref, wp2_ref, bp1_ref, bp2_ref, out_ref):
    f32 = jnp.float32
    a1 = wall_ref[0:F, :]
    a2 = wall_ref[F:4 * F, :]
    tsmall = jnp.dot(wsm_ref[...], a2, preferred_element_type=f32)
    g = g_ref[0]
    a = a_ref[0]
    o = o_ref[0]
    cols = lax.broadcasted_iota(jnp.int32, (TB, 32), 1)
    sh = ((cols == o) | (cols == a + 21) | (cols == g + 28)).astype(f32)
    u_tmp = (jnp.dot(uid_ref[...], a1, preferred_element_type=f32)
             + jnp.dot(sh, tsmall, preferred_element_type=f32)
             + ball_ref[...])
    c1 = wcu_ref[0:F, :]
    c2 = wcu_ref[F:2 * F, :] * (1.0 / NB)
    uf = (jnp.dot(u_tmp, c1, preferred_element_type=f32)
          + jnp.dot(us_ref[...], c2, preferred_element_type=f32)
          + bcu_ref[...])
    d1 = wci_ref[0:F, :]
    d2 = wci_ref[F:2 * F, :] * (1.0 / NB)
    itf = (jnp.dot(item_ref[...], d1, preferred_element_type=f32)
           + jnp.dot(is_ref[...], d2, preferred_element_type=f32)
           + bci_ref[...])
    pvec = jnp.sum(wp1t_ref[...] * wp2_ref[...], axis=0)
    cconst = jnp.sum(bp1_ref[...] * wp2_ref[...]) + bp2_ref[0, 0]
    out_ref[0] = jnp.sum(uf * itf * pvec, axis=1, keepdims=True) + cconst


def _tc_mlp(uid_e, item_e, usage_s, isage_s, g3, a3, o3, w_all, wsm, w_cu,
            w_ci, b_all, b_cu, b_ci, wp1t, wp2, bp1, bp2):
    emb_spec = pl.BlockSpec((TB, F), lambda i: (i, 0))
    idx_spec = pl.BlockSpec((1, TB, 1), lambda i: (i, 0, 0))

    def full(x):
        r = len(x.shape)
        return pl.BlockSpec(x.shape, lambda i, _r=r: (0,) * _r)

    return pl.pallas_call(
        _tc_mlp_body,
        grid=(NT,),
        in_specs=[emb_spec, emb_spec, emb_spec, emb_spec,
                  idx_spec, idx_spec, idx_spec,
                  full(w_all), full(wsm), full(w_cu), full(w_ci),
                  full(b_all), full(b_cu), full(b_ci),
                  full(wp1t), full(wp2), full(bp1), full(bp2)],
        out_specs=pl.BlockSpec((1, TB, 1), lambda i: (i, 0, 0)),
        out_shape=jax.ShapeDtypeStruct((NT, TB, 1), jnp.float32),
    )(uid_e, item_e, usage_s, isage_s, g3, a3, o3, w_all, wsm, w_cu, w_ci,
      b_all, b_cu, b_ci, wp1t, wp2, bp1, bp2)


def kernel(user, item, user_gender, user_age, user_occupation, user_neighbor,
           item_neighbor, W_user_gmf, W_item_gmf, W_user_sage, W_item_sage,
           W_gender, W_age, W_occ, W_all, b_all, W_cu, b_cu, W_ci, b_ci,
           W_p1, b_p1, W_p2, b_p2):
    i32 = jnp.int32
    user = user.astype(i32)
    item = item.astype(i32)
    un_t = user_neighbor.astype(i32).reshape(NW, BPW, NB).transpose(0, 2, 1)
    in_t = item_neighbor.astype(i32).reshape(NW, BPW, NB).transpose(0, 2, 1)

    uid_e, item_e, usage_s, isage_s = _sc_gather()(
        user, item, un_t, in_t, W_user_gmf, W_item_gmf, W_user_sage,
        W_item_sage)

    wsm = jnp.zeros((32, 3 * F), jnp.float32)
    wsm = wsm.at[0:21, 0:F].set(W_occ)
    wsm = wsm.at[21:28, F:2 * F].set(W_age)
    wsm = wsm.at[28:30, 2 * F:3 * F].set(W_gender)

    g3 = user_gender.astype(i32).reshape(NT, TB, 1)
    a3 = user_age.astype(i32).reshape(NT, TB, 1)
    o3 = user_occupation.astype(i32).reshape(NT, TB, 1)

    pred = _tc_mlp(uid_e, item_e, usage_s, isage_s, g3, a3, o3,
                   W_all, wsm, W_cu, W_ci,
                   b_all.reshape(1, F), b_cu.reshape(1, F), b_ci.reshape(1, F),
                   W_p1.T, W_p2, b_p1.reshape(8, 1), b_p2.reshape(1, 1))
    return pred.reshape(-1)


# speedup vs baseline: 3.7115x; 1.2304x over previous
import functools

import jax
import jax.numpy as jnp
from jax import lax
from jax.experimental import pallas as pl
from jax.experimental.pallas import tpu as pltpu
from jax.experimental.pallas import tpu_sc as plsc

B = 4096
F = 128
NB = 20
NC = 2
NS = 16
NW = NC * NS
BPW = B // NW
TB = 512
NT = B // TB


CH = 4
CIDX = CH * NB
NCH = BPW // CH
RING = 4


def _sc_gather_body(user, item, un_flat, in_flat, w_user_gmf, w_item_gmf,
                    w_user_sage, w_item_sage, uid_out, item_out, usage_out,
                    isage_out, idq_u, idq_i, idxf, b0, b1, b2, b3, acc, idb,
                    s0, s1, s2, s3, semw):
    wid = lax.axis_index("s") * NC + lax.axis_index("c")
    base = wid * BPW
    bufs = (b0, b1, b2, b3)
    sems = (s0, s1, s2, s3)

    pltpu.sync_copy(user.at[pl.ds(base, BPW)], idq_u)
    pltpu.sync_copy(item.at[pl.ds(base, BPW)], idq_i)
    cp_idu = pltpu.async_copy(w_user_gmf.at[idq_u], idb.at[pl.ds(0, BPW)], semw)
    cp_idi = pltpu.async_copy(w_item_gmf.at[idq_i], idb.at[pl.ds(BPW, BPW)], semw)

    def fire(table, c, q):
        return pltpu.async_copy(
            table.at[idxf.at[pl.ds(c * CIDX, CIDX)]], bufs[q], sems[q])

    def side_start(idx_hbm, table):
        pltpu.sync_copy(idx_hbm.at[pl.ds(base * NB, BPW * NB)], idxf)
        for q in range(RING):
            fire(table, q, q)

    def side_run(table, out):
        def jbody(j, carry):
            for q in range(RING):
                c = RING * j + q
                pltpu.make_async_copy(
                    table.at[idxf.at[pl.ds(c * CIDX, CIDX)]],
                    bufs[q], sems[q]).wait()
                b = bufs[q]
                for br in range(CH):
                    row0 = NB * br
                    cs = tuple(b[row0, pl.ds(16 * v, 16)]
                               for v in range(F // 16))

                    def nbody(n, cs, b=b, row0=row0):
                        return tuple(cs[v] + b[row0 + n, pl.ds(16 * v, 16)]
                                     for v in range(F // 16))

                    cs = lax.fori_loop(1, NB, nbody, cs)
                    arow = CH * c + br
                    for v in range(F // 16):
                        acc[arow, pl.ds(16 * v, 16)] = cs[v]

                @pl.when(j < (NCH // RING) - 1)
                def _():
                    fire(table, c + RING, q)
            return carry

        lax.fori_loop(0, NCH // RING, jbody, 0)
        pltpu.sync_copy(acc, out.at[pl.ds(base, BPW)])

    side_start(un_flat, w_item_sage)
    cp_idu.wait()
    cp_idi.wait()
    cp_wu = pltpu.async_copy(idb.at[pl.ds(0, BPW)],
                             uid_out.at[pl.ds(base, BPW)], semw)
    cp_wi = pltpu.async_copy(idb.at[pl.ds(BPW, BPW)],
                             item_out.at[pl.ds(base, BPW)], semw)
    side_run(w_item_sage, usage_out)
    side_start(in_flat, w_user_sage)
    side_run(w_user_sage, isage_out)
    cp_wu.wait()
    cp_wi.wait()


@functools.lru_cache(maxsize=1)
def _sc_gather():
    return pl.kernel(
        _sc_gather_body,
        out_type=(
            jax.ShapeDtypeStruct((B, F), jnp.float32),
            jax.ShapeDtypeStruct((B, F), jnp.float32),
            jax.ShapeDtypeStruct((B, F), jnp.float32),
            jax.ShapeDtypeStruct((B, F), jnp.float32),
        ),
        mesh=plsc.VectorSubcoreMesh(core_axis_name="c", subcore_axis_name="s"),
        scratch_types=[
            pltpu.VMEM((BPW,), jnp.int32),
            pltpu.VMEM((BPW,), jnp.int32),
            pltpu.VMEM((BPW * NB,), jnp.int32),
            pltpu.VMEM((CIDX, F), jnp.float32),
            pltpu.VMEM((CIDX, F), jnp.float32),
            pltpu.VMEM((CIDX, F), jnp.float32),
            pltpu.VMEM((CIDX, F), jnp.float32),
            pltpu.VMEM((BPW, F), jnp.float32),
            pltpu.VMEM((2 * BPW, F), jnp.float32),
            pltpu.SemaphoreType.DMA,
            pltpu.SemaphoreType.DMA,
            pltpu.SemaphoreType.DMA,
            pltpu.SemaphoreType.DMA,
            pltpu.SemaphoreType.DMA,
        ],
    )


def _tc_mlp_body(uid_ref, item_ref, us_ref, is_ref, g_ref, a_ref, o_ref,
                 wall_ref, wsm_ref, wcu_ref, wci_ref, ball_ref, bcu_ref,
                 bci_ref, wp1t_ref, wp2_ref, bp1_ref, bp2_ref, out_ref):
    f32 = jnp.float32
    a1 = wall_ref[0:F, :]
    a2 = wall_ref[F:4 * F, :]
    tsmall = jnp.dot(wsm_ref[...], a2, preferred_element_type=f32)
    g = g_ref[0]
    a = a_ref[0]
    o = o_ref[0]
    cols = lax.broadcasted_iota(jnp.int32, (TB, 32), 1)
    sh = ((cols == o) | (cols == a + 21) | (cols == g + 28)).astype(f32)
    u_tmp = (jnp.dot(uid_ref[...], a1, preferred_element_type=f32)
             + jnp.dot(sh, tsmall, preferred_element_type=f32)
             + ball_ref[...])
    c1 = wcu_ref[0:F, :]
    c2 = wcu_ref[F:2 * F, :] * (1.0 / NB)
    uf = (jnp.dot(u_tmp, c1, preferred_element_type=f32)
          + jnp.dot(us_ref[...], c2, preferred_element_type=f32)
          + bcu_ref[...])
    d1 = wci_ref[0:F, :]
    d2 = wci_ref[F:2 * F, :] * (1.0 / NB)
    itf = (jnp.dot(item_ref[...], d1, preferred_element_type=f32)
           + jnp.dot(is_ref[...], d2, preferred_element_type=f32)
           + bci_ref[...])
    pvec = jnp.sum(wp1t_ref[...] * wp2_ref[...], axis=0)
    cconst = jnp.sum(bp1_ref[...] * wp2_ref[...]) + bp2_ref[0, 0]
    out_ref[0] = jnp.sum(uf * itf * pvec, axis=1, keepdims=True) + cconst


def _tc_mlp(uid_e, item_e, usage_s, isage_s, g3, a3, o3, w_all, wsm, w_cu,
            w_ci, b_all, b_cu, b_ci, wp1t, wp2, bp1, bp2):
    emb_spec = pl.BlockSpec((TB, F), lambda i: (i, 0))
    idx_spec = pl.BlockSpec((1, TB, 1), lambda i: (i, 0, 0))

    def full(x):
        r = len(x.shape)
        return pl.BlockSpec(x.shape, lambda i, _r=r: (0,) * _r)

    return pl.pallas_call(
        _tc_mlp_body,
        grid=(NT,),
        in_specs=[emb_spec, emb_spec, emb_spec, emb_spec,
                  idx_spec, idx_spec, idx_spec,
                  full(w_all), full(wsm), full(w_cu), full(w_ci),
                  full(b_all), full(b_cu), full(b_ci),
                  full(wp1t), full(wp2), full(bp1), full(bp2)],
        out_specs=pl.BlockSpec((1, TB, 1), lambda i: (i, 0, 0)),
        out_shape=jax.ShapeDtypeStruct((NT, TB, 1), jnp.float32),
    )(uid_e, item_e, usage_s, isage_s, g3, a3, o3, w_all, wsm, w_cu, w_ci,
      b_all, b_cu, b_ci, wp1t, wp2, bp1, bp2)


def kernel(user, item, user_gender, user_age, user_occupation, user_neighbor,
           item_neighbor, W_user_gmf, W_item_gmf, W_user_sage, W_item_sage,
           W_gender, W_age, W_occ, W_all, b_all, W_cu, b_cu, W_ci, b_ci,
           W_p1, b_p1, W_p2, b_p2):
    i32 = jnp.int32
    user = user.astype(i32)
    item = item.astype(i32)
    un_flat = user_neighbor.astype(i32).reshape(-1)
    in_flat = item_neighbor.astype(i32).reshape(-1)

    uid_e, item_e, usage_s, isage_s = _sc_gather()(
        user, item, un_flat, in_flat, W_user_gmf, W_item_gmf, W_user_sage,
        W_item_sage)

    wsm = jnp.zeros((32, 3 * F), jnp.float32)
    wsm = wsm.at[0:21, 0:F].set(W_occ)
    wsm = wsm.at[21:28, F:2 * F].set(W_age)
    wsm = wsm.at[28:30, 2 * F:3 * F].set(W_gender)

    g3 = user_gender.astype(i32).reshape(NT, TB, 1)
    a3 = user_age.astype(i32).reshape(NT, TB, 1)
    o3 = user_occupation.astype(i32).reshape(NT, TB, 1)

    pred = _tc_mlp(uid_e, item_e, usage_s, isage_s, g3, a3, o3,
                   W_all, wsm, W_cu, W_ci,
                   b_all.reshape(1, F), b_cu.reshape(1, F), b_ci.reshape(1, F),
                   W_p1.T, W_p2, b_p1.reshape(8, 1), b_p2.reshape(1, 1))
    return pred.reshape(-1)


# speedup vs baseline: 3.7605x; 1.0132x over previous
import functools

import jax
import jax.numpy as jnp
from jax import lax
from jax.experimental import pallas as pl
from jax.experimental.pallas import tpu as pltpu
from jax.experimental.pallas import tpu_sc as plsc

B = 4096
F = 128
NB = 20
NC = 2
NS = 16
NW = NC * NS
BPW = B // NW
TB = 1024
NT = B // TB


CH = 4
CIDX = CH * NB
NCH = BPW // CH
RING = 4


def _sc_gather_body(user, item, un_flat, in_flat, w_user_gmf, w_item_gmf,
                    w_user_sage, w_item_sage, uid_out, item_out, usage_out,
                    isage_out, idq_u, idq_i, idxf, idxf2, b0, b1, b2, b3, acc,
                    idb, s0, s1, s2, s3, semw, semx):
    wid = lax.axis_index("s") * NC + lax.axis_index("c")
    base = wid * BPW
    bufs = (b0, b1, b2, b3)
    sems = (s0, s1, s2, s3)

    pltpu.sync_copy(user.at[pl.ds(base, BPW)], idq_u)
    pltpu.sync_copy(item.at[pl.ds(base, BPW)], idq_i)
    cp_idu = pltpu.async_copy(w_user_gmf.at[idq_u], idb.at[pl.ds(0, BPW)], semw)
    cp_idi = pltpu.async_copy(w_item_gmf.at[idq_i], idb.at[pl.ds(BPW, BPW)], semw)

    def fire(table, ixf, c, q):
        return pltpu.async_copy(
            table.at[ixf.at[pl.ds(c * CIDX, CIDX)]], bufs[q], sems[q])

    def side_run(table, ixf, out):
        for q in range(RING):
            fire(table, ixf, q, q)

        def jbody(j, carry):
            for q in range(RING):
                c = RING * j + q
                pltpu.make_async_copy(
                    table.at[ixf.at[pl.ds(c * CIDX, CIDX)]],
                    bufs[q], sems[q]).wait()
                b = bufs[q]
                for br in range(CH):
                    row0 = NB * br
                    cs = tuple(b[row0, pl.ds(16 * v, 16)]
                               for v in range(F // 16))

                    def nbody(m, cs, b=b, row0=row0):
                        n = 2 * m + 1
                        return tuple(cs[v] + b[row0 + n, pl.ds(16 * v, 16)]
                                     + b[row0 + n + 1, pl.ds(16 * v, 16)]
                                     for v in range(F // 16))

                    cs = lax.fori_loop(0, (NB - 2) // 2, nbody, cs)
                    arow = CH * c + br
                    for v in range(F // 16):
                        acc[arow, pl.ds(16 * v, 16)] = (
                            cs[v] + b[row0 + NB - 1, pl.ds(16 * v, 16)])

                @pl.when(j < (NCH // RING) - 1)
                def _():
                    fire(table, ixf, c + RING, q)
            return carry

        lax.fori_loop(0, NCH // RING, jbody, 0)
        pltpu.sync_copy(acc, out.at[pl.ds(base, BPW)])

    pltpu.sync_copy(un_flat.at[pl.ds(base * NB, BPW * NB)], idxf)
    cp_x2 = pltpu.async_copy(in_flat.at[pl.ds(base * NB, BPW * NB)], idxf2,
                             semx)
    cp_idu.wait()
    cp_idi.wait()
    cp_wu = pltpu.async_copy(idb.at[pl.ds(0, BPW)],
                             uid_out.at[pl.ds(base, BPW)], semw)
    cp_wi = pltpu.async_copy(idb.at[pl.ds(BPW, BPW)],
                             item_out.at[pl.ds(base, BPW)], semw)
    side_run(w_item_sage, idxf, usage_out)
    cp_x2.wait()
    side_run(w_user_sage, idxf2, isage_out)
    cp_wu.wait()
    cp_wi.wait()


@functools.lru_cache(maxsize=1)
def _sc_gather():
    return pl.kernel(
        _sc_gather_body,
        out_type=(
            jax.ShapeDtypeStruct((B, F), jnp.float32),
            jax.ShapeDtypeStruct((B, F), jnp.float32),
            jax.ShapeDtypeStruct((B, F), jnp.float32),
            jax.ShapeDtypeStruct((B, F), jnp.float32),
        ),
        mesh=plsc.VectorSubcoreMesh(core_axis_name="c", subcore_axis_name="s"),
        scratch_types=[
            pltpu.VMEM((BPW,), jnp.int32),
            pltpu.VMEM((BPW,), jnp.int32),
            pltpu.VMEM((BPW * NB,), jnp.int32),
            pltpu.VMEM((BPW * NB,), jnp.int32),
            pltpu.VMEM((CIDX, F), jnp.float32),
            pltpu.VMEM((CIDX, F), jnp.float32),
            pltpu.VMEM((CIDX, F), jnp.float32),
            pltpu.VMEM((CIDX, F), jnp.float32),
            pltpu.VMEM((BPW, F), jnp.float32),
            pltpu.VMEM((2 * BPW, F), jnp.float32),
            pltpu.SemaphoreType.DMA,
            pltpu.SemaphoreType.DMA,
            pltpu.SemaphoreType.DMA,
            pltpu.SemaphoreType.DMA,
            pltpu.SemaphoreType.DMA,
            pltpu.SemaphoreType.DMA,
        ],
    )


def _tc_mlp_body(uid_ref, item_ref, us_ref, is_ref, g_ref, a_ref, o_ref,
                 wall_ref, wsm_ref, wcu_ref, wci_ref, ball_ref, bcu_ref,
                 bci_ref, wp1t_ref, wp2_ref, bp1_ref, bp2_ref, out_ref):
    f32 = jnp.float32
    a1 = wall_ref[0:F, :]
    a2 = wall_ref[F:4 * F, :]
    tsmall = jnp.dot(wsm_ref[...], a2, preferred_element_type=f32)
    g = jnp.reshape(g_ref[0], (TB, 1))
    a = jnp.reshape(a_ref[0], (TB, 1))
    o = jnp.reshape(o_ref[0], (TB, 1))
    cols = lax.broadcasted_iota(jnp.int32, (TB, 32), 1)
    sh = ((cols == o) | (cols == a + 21) | (cols == g + 28)).astype(f32)
    u_tmp = (jnp.dot(uid_ref[...], a1, preferred_element_type=f32)
             + jnp.dot(sh, tsmall, preferred_element_type=f32)
             + ball_ref[...])
    c1 = wcu_ref[0:F, :]
    c2 = wcu_ref[F:2 * F, :] * (1.0 / NB)
    uf = (jnp.dot(u_tmp, c1, preferred_element_type=f32)
          + jnp.dot(us_ref[...], c2, preferred_element_type=f32)
          + bcu_ref[...])
    d1 = wci_ref[0:F, :]
    d2 = wci_ref[F:2 * F, :] * (1.0 / NB)
    itf = (jnp.dot(item_ref[...], d1, preferred_element_type=f32)
           + jnp.dot(is_ref[...], d2, preferred_element_type=f32)
           + bci_ref[...])
    pvec = jnp.sum(wp1t_ref[...] * wp2_ref[...], axis=0)
    cconst = jnp.sum(bp1_ref[...] * wp2_ref[...]) + bp2_ref[0, 0]
    out_ref[0, 0, :] = jnp.sum(uf * itf * pvec, axis=1) + cconst


def _tc_mlp(uid_e, item_e, usage_s, isage_s, g3, a3, o3, w_all, wsm, w_cu,
            w_ci, b_all, b_cu, b_ci, wp1t, wp2, bp1, bp2):
    emb_spec = pl.BlockSpec((TB, F), lambda i: (i, 0))
    idx_spec = pl.BlockSpec((1, 1, TB), lambda i: (i, 0, 0))

    def full(x):
        r = len(x.shape)
        return pl.BlockSpec(x.shape, lambda i, _r=r: (0,) * _r)

    return pl.pallas_call(
        _tc_mlp_body,
        grid=(NT,),
        in_specs=[emb_spec, emb_spec, emb_spec, emb_spec,
                  idx_spec, idx_spec, idx_spec,
                  full(w_all), full(wsm), full(w_cu), full(w_ci),
                  full(b_all), full(b_cu), full(b_ci),
                  full(wp1t), full(wp2), full(bp1), full(bp2)],
        out_specs=pl.BlockSpec((1, 1, TB), lambda i: (i, 0, 0)),
        out_shape=jax.ShapeDtypeStruct((NT, 1, TB), jnp.float32),
    )(uid_e, item_e, usage_s, isage_s, g3, a3, o3, w_all, wsm, w_cu, w_ci,
      b_all, b_cu, b_ci, wp1t, wp2, bp1, bp2)


def kernel(user, item, user_gender, user_age, user_occupation, user_neighbor,
           item_neighbor, W_user_gmf, W_item_gmf, W_user_sage, W_item_sage,
           W_gender, W_age, W_occ, W_all, b_all, W_cu, b_cu, W_ci, b_ci,
           W_p1, b_p1, W_p2, b_p2):
    i32 = jnp.int32
    user = user.astype(i32)
    item = item.astype(i32)
    un_flat = user_neighbor.astype(i32).reshape(-1)
    in_flat = item_neighbor.astype(i32).reshape(-1)

    uid_e, item_e, usage_s, isage_s = _sc_gather()(
        user, item, un_flat, in_flat, W_user_gmf, W_item_gmf, W_user_sage,
        W_item_sage)

    wsm = jnp.zeros((32, 3 * F), jnp.float32)
    wsm = wsm.at[0:21, 0:F].set(W_occ)
    wsm = wsm.at[21:28, F:2 * F].set(W_age)
    wsm = wsm.at[28:30, 2 * F:3 * F].set(W_gender)

    g3 = user_gender.astype(i32).reshape(NT, 1, TB)
    a3 = user_age.astype(i32).reshape(NT, 1, TB)
    o3 = user_occupation.astype(i32).reshape(NT, 1, TB)

    pred = _tc_mlp(uid_e, item_e, usage_s, isage_s, g3, a3, o3,
                   W_all, wsm, W_cu, W_ci,
                   b_all.reshape(1, F), b_cu.reshape(1, F), b_ci.reshape(1, F),
                   W_p1.T, W_p2, b_p1.reshape(8, 1), b_p2.reshape(1, 1))
    return pred.reshape(-1)


# speedup vs baseline: 4.1983x; 1.1164x over previous
import functools

import jax
import jax.numpy as jnp
from jax import lax
from jax.experimental import pallas as pl
from jax.experimental.pallas import tpu as pltpu
from jax.experimental.pallas import tpu_sc as plsc

B = 4096
F = 128
NB = 20
NC = 2
NS = 16
NW = NC * NS
BPW = B // NW
TB = 1024
NT = B // TB


CH = 4
CIDX = CH * NB
NCH = BPW // CH
RING = 4


def _sc_gather_body(user, item, un_flat, in_flat, w_user_gmf, w_item_gmf,
                    w_user_sage, w_item_sage, uid_out, item_out, usage_out,
                    isage_out, idq_u, idq_i, idxf, idxf2, b0, b1, b2, b3, acc,
                    idb, s0, s1, s2, s3, semw, semx):
    wid = lax.axis_index("s") * NC + lax.axis_index("c")
    base = wid * BPW
    bufs = (b0, b1, b2, b3)
    sems = (s0, s1, s2, s3)

    pltpu.sync_copy(user.at[pl.ds(base, BPW)], idq_u)
    pltpu.sync_copy(item.at[pl.ds(base, BPW)], idq_i)
    cp_idu = pltpu.async_copy(w_user_gmf.at[idq_u], idb.at[pl.ds(0, BPW)], semw)
    cp_idi = pltpu.async_copy(w_item_gmf.at[idq_i], idb.at[pl.ds(BPW, BPW)], semw)

    def fire(table, ixf, c, q):
        return pltpu.async_copy(
            table.at[ixf.at[pl.ds(c * CIDX, CIDX)]], bufs[q], sems[q])

    pltpu.sync_copy(un_flat.at[pl.ds(base * NB, BPW * NB)], idxf)
    cp_x2 = pltpu.async_copy(in_flat.at[pl.ds(base * NB, BPW * NB)], idxf2,
                             semx)
    for q in range(RING):
        fire(w_item_sage, idxf, q, q)
    cp_idu.wait()
    cp_idi.wait()
    cp_wu = pltpu.async_copy(idb.at[pl.ds(0, BPW)],
                             uid_out.at[pl.ds(base, BPW)], semw)
    cp_wi = pltpu.async_copy(idb.at[pl.ds(BPW, BPW)],
                             item_out.at[pl.ds(base, BPW)], semw)
    cp_x2.wait()

    def jbody(j, carry):
        for q in range(RING):
            c = RING * j + q
            pltpu.make_async_copy(
                w_item_sage.at[idxf.at[pl.ds(0, CIDX)]], bufs[q],
                sems[q]).wait()
            b = bufs[q]
            arow0 = CH * c - jnp.where(c >= NCH, CH * NCH, 0)
            for br in range(CH):
                row0 = NB * br
                cs = tuple(b[row0, pl.ds(16 * v, 16)]
                           + b[row0 + 1, pl.ds(16 * v, 16)]
                           for v in range(F // 16))

                def nbody(m, cs, b=b, row0=row0):
                    r = row0 + 2 * m
                    return tuple(cs[v] + b[r, pl.ds(16 * v, 16)]
                                 + b[r + 1, pl.ds(16 * v, 16)]
                                 for v in range(F // 16))

                cs = lax.fori_loop(1, NB // 2, nbody, cs)
                for v in range(F // 16):
                    acc[arow0 + br, pl.ds(16 * v, 16)] = cs[v]

            cn = c + RING

            @pl.when(cn < NCH)
            def _():
                fire(w_item_sage, idxf, cn, q)

            @pl.when((cn >= NCH) & (cn < 2 * NCH))
            def _():
                fire(w_user_sage, idxf2, cn - NCH, q)

            @pl.when(c == NCH - 1)
            def _():
                pltpu.sync_copy(acc, usage_out.at[pl.ds(base, BPW)])
        return carry

    lax.fori_loop(0, 2 * NCH // RING, jbody, 0)
    pltpu.sync_copy(acc, isage_out.at[pl.ds(base, BPW)])
    cp_wu.wait()
    cp_wi.wait()


@functools.lru_cache(maxsize=1)
def _sc_gather():
    return pl.kernel(
        _sc_gather_body,
        out_type=(
            jax.ShapeDtypeStruct((B, F), jnp.float32),
            jax.ShapeDtypeStruct((B, F), jnp.float32),
            jax.ShapeDtypeStruct((B, F), jnp.float32),
            jax.ShapeDtypeStruct((B, F), jnp.float32),
        ),
        mesh=plsc.VectorSubcoreMesh(core_axis_name="c", subcore_axis_name="s"),
        scratch_types=[
            pltpu.VMEM((BPW,), jnp.int32),
            pltpu.VMEM((BPW,), jnp.int32),
            pltpu.VMEM((BPW * NB,), jnp.int32),
            pltpu.VMEM((BPW * NB,), jnp.int32),
            pltpu.VMEM((CIDX, F), jnp.float32),
            pltpu.VMEM((CIDX, F), jnp.float32),
            pltpu.VMEM((CIDX, F), jnp.float32),
            pltpu.VMEM((CIDX, F), jnp.float32),
            pltpu.VMEM((BPW, F), jnp.float32),
            pltpu.VMEM((2 * BPW, F), jnp.float32),
            pltpu.SemaphoreType.DMA,
            pltpu.SemaphoreType.DMA,
            pltpu.SemaphoreType.DMA,
            pltpu.SemaphoreType.DMA,
            pltpu.SemaphoreType.DMA,
            pltpu.SemaphoreType.DMA,
        ],
    )


def _tc_mlp_body(uid_ref, item_ref, us_ref, is_ref, g_ref, a_ref, o_ref,
                 wall_ref, wsm_ref, wcu_ref, wci_ref, ball_ref, bcu_ref,
                 bci_ref, wp1t_ref, wp2_ref, bp1_ref, bp2_ref, out_ref):
    f32 = jnp.float32
    a1 = wall_ref[0:F, :]
    a2 = wall_ref[F:4 * F, :]
    tsmall = jnp.dot(wsm_ref[...], a2, preferred_element_type=f32)
    g = jnp.reshape(g_ref[0], (TB, 1))
    a = jnp.reshape(a_ref[0], (TB, 1))
    o = jnp.reshape(o_ref[0], (TB, 1))
    cols = lax.broadcasted_iota(jnp.int32, (TB, 32), 1)
    sh = ((cols == o) | (cols == a + 21) | (cols == g + 28)).astype(f32)
    c1 = wcu_ref[0:F, :]
    c2 = wcu_ref[F:2 * F, :] * (1.0 / NB)
    e1 = jnp.dot(a1, c1, preferred_element_type=f32)
    tsc = jnp.dot(tsmall, c1, preferred_element_type=f32)
    bu = jnp.dot(ball_ref[...], c1, preferred_element_type=f32) + bcu_ref[...]
    uf = (jnp.dot(uid_ref[...], e1, preferred_element_type=f32)
          + jnp.dot(sh, tsc, preferred_element_type=f32)
          + jnp.dot(us_ref[...], c2, preferred_element_type=f32)
          + bu)
    d1 = wci_ref[0:F, :]
    d2 = wci_ref[F:2 * F, :] * (1.0 / NB)
    itf = (jnp.dot(item_ref[...], d1, preferred_element_type=f32)
           + jnp.dot(is_ref[...], d2, preferred_element_type=f32)
           + bci_ref[...])
    pvec = jnp.sum(wp1t_ref[...] * wp2_ref[...], axis=0)
    cconst = jnp.sum(bp1_ref[...] * wp2_ref[...]) + bp2_ref[0, 0]
    out_ref[...] = jnp.sum(uf * itf * pvec, axis=1) + cconst


def _tc_mlp(uid_e, item_e, usage_s, isage_s, g3, a3, o3, w_all, wsm, w_cu,
            w_ci, b_all, b_cu, b_ci, wp1t, wp2, bp1, bp2):
    emb_spec = pl.BlockSpec((TB, F), lambda i: (i, 0))
    idx_spec = pl.BlockSpec((1, 1, TB), lambda i: (i, 0, 0))

    def full(x):
        r = len(x.shape)
        return pl.BlockSpec(x.shape, lambda i, _r=r: (0,) * _r)

    return pl.pallas_call(
        _tc_mlp_body,
        grid=(NT,),
        in_specs=[emb_spec, emb_spec, emb_spec, emb_spec,
                  idx_spec, idx_spec, idx_spec,
                  full(w_all), full(wsm), full(w_cu), full(w_ci),
                  full(b_all), full(b_cu), full(b_ci),
                  full(wp1t), full(wp2), full(bp1), full(bp2)],
        out_specs=pl.BlockSpec((TB,), lambda i: (i,)),
        out_shape=jax.ShapeDtypeStruct((B,), jnp.float32),
    )(uid_e, item_e, usage_s, isage_s, g3, a3, o3, w_all, wsm, w_cu, w_ci,
      b_all, b_cu, b_ci, wp1t, wp2, bp1, bp2)


def kernel(user, item, user_gender, user_age, user_occupation, user_neighbor,
           item_neighbor, W_user_gmf, W_item_gmf, W_user_sage, W_item_sage,
           W_gender, W_age, W_occ, W_all, b_all, W_cu, b_cu, W_ci, b_ci,
           W_p1, b_p1, W_p2, b_p2):
    i32 = jnp.int32
    user = user.astype(i32)
    item = item.astype(i32)
    un_flat = user_neighbor.astype(i32).reshape(-1)
    in_flat = item_neighbor.astype(i32).reshape(-1)

    uid_e, item_e, usage_s, isage_s = _sc_gather()(
        user, item, un_flat, in_flat, W_user_gmf, W_item_gmf, W_user_sage,
        W_item_sage)

    wsm = jnp.zeros((32, 3 * F), jnp.float32)
    wsm = wsm.at[0:21, 0:F].set(W_occ)
    wsm = wsm.at[21:28, F:2 * F].set(W_age)
    wsm = wsm.at[28:30, 2 * F:3 * F].set(W_gender)

    g3 = user_gender.astype(i32).reshape(NT, 1, TB)
    a3 = user_age.astype(i32).reshape(NT, 1, TB)
    o3 = user_occupation.astype(i32).reshape(NT, 1, TB)

    pred = _tc_mlp(uid_e, item_e, usage_s, isage_s, g3, a3, o3,
                   W_all, wsm, W_cu, W_ci,
                   b_all.reshape(1, F), b_cu.reshape(1, F), b_ci.reshape(1, F),
                   W_p1.T, W_p2, b_p1.reshape(8, 1), b_p2.reshape(1, 1))
    return pred


# speedup vs baseline: 4.2739x; 1.0180x over previous
import functools

import jax
import jax.numpy as jnp
from jax import lax
from jax.experimental import pallas as pl
from jax.experimental.pallas import tpu as pltpu
from jax.experimental.pallas import tpu_sc as plsc

B = 4096
F = 128
NB = 20
NC = 2
NS = 16
NW = NC * NS
BPW = B // NW
TB = 2048
NT = B // TB


CH = 4
CIDX = CH * NB
NCH = BPW // CH
RING = 4


def _sc_gather_body(user, item, un_flat, in_flat, w_user_gmf, w_item_gmf,
                    w_user_sage, w_item_sage, uid_out, item_out, usage_out,
                    isage_out, idq_u, idq_i, idxf, idxf2, b0, b1, b2, b3, acc,
                    idb, s0, s1, s2, s3, semw, semx):
    wid = lax.axis_index("s") * NC + lax.axis_index("c")
    base = wid * BPW
    bufs = (b0, b1, b2, b3)
    sems = (s0, s1, s2, s3)

    pltpu.sync_copy(user.at[pl.ds(base, BPW)], idq_u)
    pltpu.sync_copy(item.at[pl.ds(base, BPW)], idq_i)
    cp_idu = pltpu.async_copy(w_user_gmf.at[idq_u], idb.at[pl.ds(0, BPW)], semw)
    cp_idi = pltpu.async_copy(w_item_gmf.at[idq_i], idb.at[pl.ds(BPW, BPW)], semw)

    def fire(table, ixf, c, q):
        return pltpu.async_copy(
            table.at[ixf.at[pl.ds(c * CIDX, CIDX)]], bufs[q], sems[q])

    pltpu.sync_copy(un_flat.at[pl.ds(base * NB, BPW * NB)], idxf)
    cp_x2 = pltpu.async_copy(in_flat.at[pl.ds(base * NB, BPW * NB)], idxf2,
                             semx)
    for q in range(RING):
        fire(w_item_sage, idxf, q, q)
    cp_idu.wait()
    cp_idi.wait()
    cp_wu = pltpu.async_copy(idb.at[pl.ds(0, BPW)],
                             uid_out.at[pl.ds(base, BPW)], semw)
    cp_wi = pltpu.async_copy(idb.at[pl.ds(BPW, BPW)],
                             item_out.at[pl.ds(base, BPW)], semw)
    cp_x2.wait()

    def jbody(j, carry):
        for q in range(RING):
            c = RING * j + q
            pltpu.make_async_copy(
                w_item_sage.at[idxf.at[pl.ds(0, CIDX)]], bufs[q],
                sems[q]).wait()
            b = bufs[q]
            arow0 = CH * c - jnp.where(c >= NCH, CH * NCH, 0)
            for br in range(CH):
                row0 = NB * br
                cs = tuple(b[row0, pl.ds(16 * v, 16)]
                           + b[row0 + 1, pl.ds(16 * v, 16)]
                           for v in range(F // 16))

                def nbody(m, cs, b=b, row0=row0):
                    r = row0 + 2 * m
                    return tuple(cs[v] + b[r, pl.ds(16 * v, 16)]
                                 + b[r + 1, pl.ds(16 * v, 16)]
                                 for v in range(F // 16))

                cs = lax.fori_loop(1, NB // 2, nbody, cs)
                for v in range(F // 16):
                    acc[arow0 + br, pl.ds(16 * v, 16)] = cs[v]

            cn = c + RING

            @pl.when(cn < NCH)
            def _():
                fire(w_item_sage, idxf, cn, q)

            @pl.when((cn >= NCH) & (cn < 2 * NCH))
            def _():
                fire(w_user_sage, idxf2, cn - NCH, q)

            @pl.when(c == NCH - 1)
            def _():
                pltpu.sync_copy(acc, usage_out.at[pl.ds(base, BPW)])
        return carry

    lax.fori_loop(0, 2 * NCH // RING, jbody, 0)
    pltpu.sync_copy(acc, isage_out.at[pl.ds(base, BPW)])
    cp_wu.wait()
    cp_wi.wait()


@functools.lru_cache(maxsize=1)
def _sc_gather():
    return pl.kernel(
        _sc_gather_body,
        out_type=(
            jax.ShapeDtypeStruct((B, F), jnp.float32),
            jax.ShapeDtypeStruct((B, F), jnp.float32),
            jax.ShapeDtypeStruct((B, F), jnp.float32),
            jax.ShapeDtypeStruct((B, F), jnp.float32),
        ),
        mesh=plsc.VectorSubcoreMesh(core_axis_name="c", subcore_axis_name="s"),
        scratch_types=[
            pltpu.VMEM((BPW,), jnp.int32),
            pltpu.VMEM((BPW,), jnp.int32),
            pltpu.VMEM((BPW * NB,), jnp.int32),
            pltpu.VMEM((BPW * NB,), jnp.int32),
            pltpu.VMEM((CIDX, F), jnp.float32),
            pltpu.VMEM((CIDX, F), jnp.float32),
            pltpu.VMEM((CIDX, F), jnp.float32),
            pltpu.VMEM((CIDX, F), jnp.float32),
            pltpu.VMEM((BPW, F), jnp.float32),
            pltpu.VMEM((2 * BPW, F), jnp.float32),
            pltpu.SemaphoreType.DMA,
            pltpu.SemaphoreType.DMA,
            pltpu.SemaphoreType.DMA,
            pltpu.SemaphoreType.DMA,
            pltpu.SemaphoreType.DMA,
            pltpu.SemaphoreType.DMA,
        ],
    )


def _tc_mlp_body(uid_ref, item_ref, us_ref, is_ref, g_ref, a_ref, o_ref,
                 wall_ref, wg_ref, wa_ref, wo_ref, wcu_ref, wci_ref, ball_ref,
                 bcu_ref, bci_ref, wp1t_ref, wp2_ref, bp1_ref, bp2_ref,
                 out_ref):
    f32 = jnp.float32
    a1 = wall_ref[0:F, :]
    tsmall = jnp.concatenate([
        jnp.dot(wo_ref[...], wall_ref[F:2 * F, :], preferred_element_type=f32),
        jnp.dot(wa_ref[...], wall_ref[2 * F:3 * F, :],
                preferred_element_type=f32),
        jnp.dot(wg_ref[...], wall_ref[3 * F:4 * F, :],
                preferred_element_type=f32),
        jnp.zeros((2, F), f32),
    ], axis=0)
    g = jnp.reshape(g_ref[0], (TB, 1))
    a = jnp.reshape(a_ref[0], (TB, 1))
    o = jnp.reshape(o_ref[0], (TB, 1))
    cols = lax.broadcasted_iota(jnp.int32, (TB, 32), 1)
    sh = ((cols == o) | (cols == a + 21) | (cols == g + 28)).astype(f32)
    c1 = wcu_ref[0:F, :]
    c2 = wcu_ref[F:2 * F, :] * (1.0 / NB)
    e1 = jnp.dot(a1, c1, preferred_element_type=f32)
    tsc = jnp.dot(tsmall, c1, preferred_element_type=f32)
    bu = jnp.dot(ball_ref[...], c1, preferred_element_type=f32) + bcu_ref[...]
    uf = (jnp.dot(uid_ref[...], e1, preferred_element_type=f32)
          + jnp.dot(sh, tsc, preferred_element_type=f32)
          + jnp.dot(us_ref[...], c2, preferred_element_type=f32)
          + bu)
    d1 = wci_ref[0:F, :]
    d2 = wci_ref[F:2 * F, :] * (1.0 / NB)
    itf = (jnp.dot(item_ref[...], d1, preferred_element_type=f32)
           + jnp.dot(is_ref[...], d2, preferred_element_type=f32)
           + bci_ref[...])
    pvec = jnp.sum(wp1t_ref[...] * wp2_ref[...], axis=0)
    cconst = jnp.sum(bp1_ref[...] * wp2_ref[...]) + bp2_ref[0, 0]
    out_ref[...] = jnp.sum(uf * itf * pvec, axis=1) + cconst


def _tc_mlp(uid_e, item_e, usage_s, isage_s, g3, a3, o3, w_all, w_g, w_a,
            w_o, w_cu, w_ci, b_all, b_cu, b_ci, wp1t, wp2, bp1, bp2):
    emb_spec = pl.BlockSpec((TB, F), lambda i: (i, 0))
    idx_spec = pl.BlockSpec((1, 1, TB), lambda i: (i, 0, 0))

    def full(x):
        r = len(x.shape)
        return pl.BlockSpec(x.shape, lambda i, _r=r: (0,) * _r)

    return pl.pallas_call(
        _tc_mlp_body,
        grid=(NT,),
        in_specs=[emb_spec, emb_spec, emb_spec, emb_spec,
                  idx_spec, idx_spec, idx_spec,
                  full(w_all), full(w_g), full(w_a), full(w_o),
                  full(w_cu), full(w_ci),
                  full(b_all), full(b_cu), full(b_ci),
                  full(wp1t), full(wp2), full(bp1), full(bp2)],
        out_specs=pl.BlockSpec((TB,), lambda i: (i,)),
        out_shape=jax.ShapeDtypeStruct((B,), jnp.float32),
    )(uid_e, item_e, usage_s, isage_s, g3, a3, o3, w_all, w_g, w_a, w_o,
      w_cu, w_ci, b_all, b_cu, b_ci, wp1t, wp2, bp1, bp2)


def kernel(user, item, user_gender, user_age, user_occupation, user_neighbor,
           item_neighbor, W_user_gmf, W_item_gmf, W_user_sage, W_item_sage,
           W_gender, W_age, W_occ, W_all, b_all, W_cu, b_cu, W_ci, b_ci,
           W_p1, b_p1, W_p2, b_p2):
    i32 = jnp.int32
    user = user.astype(i32)
    item = item.astype(i32)
    un_flat = user_neighbor.astype(i32).reshape(-1)
    in_flat = item_neighbor.astype(i32).reshape(-1)

    uid_e, item_e, usage_s, isage_s = _sc_gather()(
        user, item, un_flat, in_flat, W_user_gmf, W_item_gmf, W_user_sage,
        W_item_sage)

    g3 = user_gender.astype(i32).reshape(NT, 1, TB)
    a3 = user_age.astype(i32).reshape(NT, 1, TB)
    o3 = user_occupation.astype(i32).reshape(NT, 1, TB)

    pred = _tc_mlp(uid_e, item_e, usage_s, isage_s, g3, a3, o3,
                   W_all, W_gender, W_age, W_occ, W_cu, W_ci,
                   b_all.reshape(1, F), b_cu.reshape(1, F), b_ci.reshape(1, F),
                   W_p1.T, W_p2, b_p1.reshape(8, 1), b_p2.reshape(1, 1))
    return pred


# speedup vs baseline: 4.3579x; 1.0197x over previous
import functools

import jax
import jax.numpy as jnp
from jax import lax
from jax.experimental import pallas as pl
from jax.experimental.pallas import tpu as pltpu
from jax.experimental.pallas import tpu_sc as plsc

B = 4096
F = 128
NB = 20
NC = 2
NS = 16
NW = NC * NS
BPW = B // NW
TB = 2048
NT = B // TB


CH = 4
CIDX = CH * NB
NCH = BPW // CH
RING = 4


def _sc_gather_body(idx_all, w_user_gmf, w_item_gmf,
                    w_user_sage, w_item_sage, uid_out, item_out, usage_out,
                    isage_out, idq_u, idq_i, idxf, idxf2, b0, b1, b2, b3, acc,
                    idb, s0, s1, s2, s3, semw, semx):
    wid = lax.axis_index("s") * NC + lax.axis_index("c")
    base = wid * BPW
    bufs = (b0, b1, b2, b3)
    sems = (s0, s1, s2, s3)

    pltpu.sync_copy(idx_all.at[pl.ds(base, BPW)], idq_u)
    pltpu.sync_copy(idx_all.at[pl.ds(B + base, BPW)], idq_i)
    cp_idu = pltpu.async_copy(w_user_gmf.at[idq_u], idb.at[pl.ds(0, BPW)], semw)
    cp_idi = pltpu.async_copy(w_item_gmf.at[idq_i], idb.at[pl.ds(BPW, BPW)], semw)

    def fire(table, ixf, c, q):
        return pltpu.async_copy(
            table.at[ixf.at[pl.ds(c * CIDX, CIDX)]], bufs[q], sems[q])

    pltpu.sync_copy(idx_all.at[pl.ds(2 * B + base * NB, BPW * NB)], idxf)
    cp_x2 = pltpu.async_copy(
        idx_all.at[pl.ds(2 * B + B * NB + base * NB, BPW * NB)], idxf2, semx)
    for q in range(RING):
        fire(w_item_sage, idxf, q, q)
    cp_idu.wait()
    cp_idi.wait()
    cp_wu = pltpu.async_copy(idb.at[pl.ds(0, BPW)],
                             uid_out.at[pl.ds(base, BPW)], semw)
    cp_wi = pltpu.async_copy(idb.at[pl.ds(BPW, BPW)],
                             item_out.at[pl.ds(base, BPW)], semw)
    cp_x2.wait()

    def jbody(j, carry):
        for q in range(RING):
            c = RING * j + q
            pltpu.make_async_copy(
                w_item_sage.at[idxf.at[pl.ds(0, CIDX)]], bufs[q],
                sems[q]).wait()
            b = bufs[q]
            arow0 = CH * c - jnp.where(c >= NCH, CH * NCH, 0)

            def brbody(br, carry, b=b, arow0=arow0):
                row0 = NB * br
                cs = tuple(b[row0, pl.ds(16 * v, 16)]
                           + b[row0 + 1, pl.ds(16 * v, 16)]
                           for v in range(F // 16))

                def nbody(m, cs, b=b, row0=row0):
                    r = row0 + 2 * m
                    return tuple(cs[v] + b[r, pl.ds(16 * v, 16)]
                                 + b[r + 1, pl.ds(16 * v, 16)]
                                 for v in range(F // 16))

                cs = lax.fori_loop(1, NB // 2, nbody, cs)
                for v in range(F // 16):
                    acc[arow0 + br, pl.ds(16 * v, 16)] = cs[v]
                return carry

            lax.fori_loop(0, CH, brbody, 0)

            cn = c + RING

            @pl.when(cn < NCH)
            def _():
                fire(w_item_sage, idxf, cn, q)

            @pl.when((cn >= NCH) & (cn < 2 * NCH))
            def _():
                fire(w_user_sage, idxf2, cn - NCH, q)

            @pl.when(c == NCH - 1)
            def _():
                pltpu.sync_copy(acc, usage_out.at[pl.ds(base, BPW)])
        return carry

    lax.fori_loop(0, 2 * NCH // RING, jbody, 0)
    pltpu.sync_copy(acc, isage_out.at[pl.ds(base, BPW)])
    cp_wu.wait()
    cp_wi.wait()


@functools.lru_cache(maxsize=1)
def _sc_gather():
    return pl.kernel(
        _sc_gather_body,
        out_type=(
            jax.ShapeDtypeStruct((B, F), jnp.float32),
            jax.ShapeDtypeStruct((B, F), jnp.float32),
            jax.ShapeDtypeStruct((B, F), jnp.float32),
            jax.ShapeDtypeStruct((B, F), jnp.float32),
        ),
        mesh=plsc.VectorSubcoreMesh(core_axis_name="c", subcore_axis_name="s"),
        scratch_types=[
            pltpu.VMEM((BPW,), jnp.int32),
            pltpu.VMEM((BPW,), jnp.int32),
            pltpu.VMEM((BPW * NB,), jnp.int32),
            pltpu.VMEM((BPW * NB,), jnp.int32),
            pltpu.VMEM((CIDX, F), jnp.float32),
            pltpu.VMEM((CIDX, F), jnp.float32),
            pltpu.VMEM((CIDX, F), jnp.float32),
            pltpu.VMEM((CIDX, F), jnp.float32),
            pltpu.VMEM((BPW, F), jnp.float32),
            pltpu.VMEM((2 * BPW, F), jnp.float32),
            pltpu.SemaphoreType.DMA,
            pltpu.SemaphoreType.DMA,
            pltpu.SemaphoreType.DMA,
            pltpu.SemaphoreType.DMA,
            pltpu.SemaphoreType.DMA,
            pltpu.SemaphoreType.DMA,
        ],
    )


def _tc_mlp_body(uid_ref, item_ref, us_ref, is_ref, g_ref, a_ref, o_ref,
                 wall_ref, wg_ref, wa_ref, wo_ref, wcu_ref, wci_ref, ball_ref,
                 bcu_ref, bci_ref, wp1t_ref, wp2_ref, bp1_ref, bp2_ref,
                 out_ref):
    f32 = jnp.float32
    a1 = wall_ref[0:F, :]
    tsmall = jnp.concatenate([
        jnp.dot(wo_ref[...], wall_ref[F:2 * F, :], preferred_element_type=f32),
        jnp.dot(wa_ref[...], wall_ref[2 * F:3 * F, :],
                preferred_element_type=f32),
        jnp.dot(wg_ref[...], wall_ref[3 * F:4 * F, :],
                preferred_element_type=f32),
        jnp.zeros((2, F), f32),
    ], axis=0)
    g = jnp.reshape(g_ref[0], (TB, 1))
    a = jnp.reshape(a_ref[0], (TB, 1))
    o = jnp.reshape(o_ref[0], (TB, 1))
    cols = lax.broadcasted_iota(jnp.int32, (TB, 32), 1)
    sh = ((cols == o) | (cols == a + 21) | (cols == g + 28)).astype(f32)
    c1 = wcu_ref[0:F, :]
    c2 = wcu_ref[F:2 * F, :] * (1.0 / NB)
    e1 = jnp.dot(a1, c1, preferred_element_type=f32)
    tsc = jnp.dot(tsmall, c1, preferred_element_type=f32)
    bu = jnp.dot(ball_ref[...], c1, preferred_element_type=f32) + bcu_ref[...]
    uf = (jnp.dot(uid_ref[...], e1, preferred_element_type=f32)
          + jnp.dot(sh, tsc, preferred_element_type=f32)
          + jnp.dot(us_ref[...], c2, preferred_element_type=f32)
          + bu)
    d1 = wci_ref[0:F, :]
    d2 = wci_ref[F:2 * F, :] * (1.0 / NB)
    itf = (jnp.dot(item_ref[...], d1, preferred_element_type=f32)
           + jnp.dot(is_ref[...], d2, preferred_element_type=f32)
           + bci_ref[...])
    pvec = jnp.sum(wp1t_ref[...] * wp2_ref[...], axis=0)
    cconst = jnp.sum(bp1_ref[...] * wp2_ref[...]) + bp2_ref[0, 0]
    out_ref[...] = jnp.sum(uf * itf * pvec, axis=1) + cconst


def _tc_mlp(uid_e, item_e, usage_s, isage_s, g3, a3, o3, w_all, w_g, w_a,
            w_o, w_cu, w_ci, b_all, b_cu, b_ci, wp1t, wp2, bp1, bp2):
    emb_spec = pl.BlockSpec((TB, F), lambda i: (i, 0))
    idx_spec = pl.BlockSpec((1, 1, TB), lambda i: (i, 0, 0))

    def full(x):
        r = len(x.shape)
        return pl.BlockSpec(x.shape, lambda i, _r=r: (0,) * _r)

    return pl.pallas_call(
        _tc_mlp_body,
        grid=(NT,),
        in_specs=[emb_spec, emb_spec, emb_spec, emb_spec,
                  idx_spec, idx_spec, idx_spec,
                  full(w_all), full(w_g), full(w_a), full(w_o),
                  full(w_cu), full(w_ci),
                  full(b_all), full(b_cu), full(b_ci),
                  full(wp1t), full(wp2), full(bp1), full(bp2)],
        out_specs=pl.BlockSpec((TB,), lambda i: (i,)),
        out_shape=jax.ShapeDtypeStruct((B,), jnp.float32),
    )(uid_e, item_e, usage_s, isage_s, g3, a3, o3, w_all, w_g, w_a, w_o,
      w_cu, w_ci, b_all, b_cu, b_ci, wp1t, wp2, bp1, bp2)


def kernel(user, item, user_gender, user_age, user_occupation, user_neighbor,
           item_neighbor, W_user_gmf, W_item_gmf, W_user_sage, W_item_sage,
           W_gender, W_age, W_occ, W_all, b_all, W_cu, b_cu, W_ci, b_ci,
           W_p1, b_p1, W_p2, b_p2):
    i32 = jnp.int32
    user = user.astype(i32)
    item = item.astype(i32)
    un_flat = user_neighbor.astype(i32).reshape(-1)
    in_flat = item_neighbor.astype(i32).reshape(-1)

    idx_all = jnp.concatenate([user, item, un_flat, in_flat])
    uid_e, item_e, usage_s, isage_s = _sc_gather()(
        idx_all, W_user_gmf, W_item_gmf, W_user_sage, W_item_sage)

    g3 = user_gender.astype(i32).reshape(NT, 1, TB)
    a3 = user_age.astype(i32).reshape(NT, 1, TB)
    o3 = user_occupation.astype(i32).reshape(NT, 1, TB)

    pred = _tc_mlp(uid_e, item_e, usage_s, isage_s, g3, a3, o3,
                   W_all, W_gender, W_age, W_occ, W_cu, W_ci,
                   b_all.reshape(1, F), b_cu.reshape(1, F), b_ci.reshape(1, F),
                   W_p1.T, W_p2, b_p1.reshape(8, 1), b_p2.reshape(1, 1))
    return pred
